# baseline (speedup 1.0000x reference)
import functools

import jax
import jax.numpy as jnp
from jax import lax
from jax.experimental import pallas as pl
from jax.experimental.pallas import tpu as pltpu
from jax.experimental.pallas import tpu_sc as plsc

_INFO = plsc.get_sparse_core_info()
_NC, _NS, _L = _INFO.num_cores, _INFO.num_subcores, _INFO.num_lanes
_NW = _NC * _NS

_B = 16384
_D = 64
_BPW = _B // _NW
_CHUNK = 128
_NCHUNK = _BPW // _CHUNK


def _sc_partials(ent1, ent2, rel1, rel2, h, t, r):
    mesh = plsc.VectorSubcoreMesh(core_axis_name="c", subcore_axis_name="s")

    @functools.partial(
        pl.kernel,
        out_type=jax.ShapeDtypeStruct((_B, _L), jnp.float32),
        mesh=mesh,
        scratch_types=[
            pltpu.VMEM((_BPW,), jnp.int32),
            pltpu.VMEM((_BPW,), jnp.int32),
            pltpu.VMEM((_BPW,), jnp.int32),
            pltpu.VMEM((_CHUNK, _D), jnp.float32),
            pltpu.VMEM((_CHUNK, _D), jnp.float32),
            pltpu.VMEM((_CHUNK, _D), jnp.float32),
            pltpu.VMEM((_CHUNK, _D), jnp.float32),
            pltpu.VMEM((_CHUNK, _D), jnp.float32),
            pltpu.VMEM((_CHUNK, _D), jnp.float32),
            pltpu.VMEM((_CHUNK, _L), jnp.float32),
            pltpu.SemaphoreType.DMA,
        ],
        compiler_params=pltpu.CompilerParams(use_tc_tiling_on_sc=False),
    )
    def sc_kernel(ent1_h, ent2_h, rel1_h, rel2_h, h_h, t_h, r_h, out_h,
                  hv, tv, rv, a1, a2, b1, b2, c1, c2, ps, sem):
        wid = lax.axis_index("s") * _NC + lax.axis_index("c")
        base = wid * _BPW

        pltpu.sync_copy(h_h.at[pl.ds(base, _BPW)], hv)
        pltpu.sync_copy(t_h.at[pl.ds(base, _BPW)], tv)
        pltpu.sync_copy(r_h.at[pl.ds(base, _BPW)], rv)

        for ci in range(_NCHUNK):
            sl = pl.ds(ci * _CHUNK, _CHUNK)
            d1 = pltpu.async_copy(ent1_h.at[hv.at[sl]], a1, sem)
            d2 = pltpu.async_copy(ent2_h.at[hv.at[sl]], a2, sem)
            d3 = pltpu.async_copy(ent1_h.at[tv.at[sl]], b1, sem)
            d4 = pltpu.async_copy(ent2_h.at[tv.at[sl]], b2, sem)
            d5 = pltpu.async_copy(rel1_h.at[rv.at[sl]], c1, sem)
            d6 = pltpu.async_copy(rel2_h.at[rv.at[sl]], c2, sem)
            d1.wait(); d2.wait(); d3.wait(); d4.wait(); d5.wait(); d6.wait()

            def body(s, carry):
                acc = jnp.zeros((_L,), jnp.float32)
                for k in range(_D // _L):
                    ksl = pl.ds(k * _L, _L)
                    x1 = a1[s, ksl]
                    x2 = a2[s, ksl]
                    y1 = b1[s, ksl]
                    y2 = b2[s, ksl]
                    w1 = c1[s, ksl]
                    w2 = c2[s, ksl]
                    acc = acc + y1 * (x1 * w1 - x2 * w2) + y2 * (x2 * w1 + x1 * w2)
                ps[s] = acc
                return carry

            lax.fori_loop(0, _CHUNK, body, 0, unroll=2)
            pltpu.sync_copy(ps, out_h.at[pl.ds(base + ci * _CHUNK, _CHUNK)])

    return sc_kernel(ent1, ent2, rel1, rel2, h, t, r)


def _tc_loss(p2, yneg_rep):
    rows = _B // 8

    def tc_kernel(p_ref, y_ref, o_ref):
        t = p_ref[...] * y_ref[...]
        li = lax.broadcasted_iota(jnp.int32, (128, 128), 0)
        ci = lax.broadcasted_iota(jnp.int32, (128, 128), 1)
        fold = (li // _L == ci // _L).astype(jnp.float32)
        z = jnp.dot(t, fold, preferred_element_type=jnp.float32)
        sp = jnp.maximum(z, 0.0) + jnp.log1p(jnp.exp(-jnp.abs(z)))
        o_ref[0, 0] = jnp.sum(sp) * (1.0 / (_L * _B))

    return pl.pallas_call(
        tc_kernel,
        out_shape=jax.ShapeDtypeStruct((1, 1), jnp.float32),
        out_specs=pl.BlockSpec(memory_space=pltpu.SMEM),
    )(p2, yneg_rep)


def kernel(ent1, ent2, rel1, rel2, h, t, r, y):
    partials = _sc_partials(ent1, ent2, rel1, rel2, h, t, r)
    p2 = partials.reshape(_B // 8, 8 * _L)
    yneg_rep = jnp.repeat(-y, _L).reshape(_B // 8, 8 * _L)
    loss = _tc_loss(p2, yneg_rep)
    return loss[0, 0]


# speedup vs baseline: 1.5233x; 1.5233x over previous
import functools

import jax
import jax.numpy as jnp
from jax import lax
from jax.experimental import pallas as pl
from jax.experimental.pallas import tpu as pltpu
from jax.experimental.pallas import tpu_sc as plsc

_INFO = plsc.get_sparse_core_info()
_NC, _NS, _L = _INFO.num_cores, _INFO.num_subcores, _INFO.num_lanes
_NW = _NC * _NS

_B = 16384
_D = 64
_BPW = _B // _NW
_NBUF = 8
_NGRP = _BPW // _NBUF


def _sc_partials(ent1, ent2, rel1, rel2, h, t, r):
    mesh = plsc.VectorSubcoreMesh(core_axis_name="c", subcore_axis_name="s")

    @functools.partial(
        pl.kernel,
        out_type=jax.ShapeDtypeStruct((_B // _NBUF, _NBUF * _L), jnp.float32),
        mesh=mesh,
        scratch_types=[
            pltpu.VMEM((_BPW + _L,), jnp.int32),
            pltpu.VMEM((_BPW + _L,), jnp.int32),
            pltpu.VMEM((_BPW + _L,), jnp.int32),
            pltpu.VMEM((_NBUF, 6, _D), jnp.float32),
            pltpu.VMEM((_NGRP, _NBUF * _L), jnp.float32),
            pltpu.SemaphoreType.DMA((_NBUF,)),
        ],
    )
    def sc_kernel(ent1_h, ent2_h, rel1_h, rel2_h, h_h, t_h, r_h, out_h,
                  hv, tv, rv, ring, ps, sem):
        wid = lax.axis_index("s") * _NC + lax.axis_index("c")
        base = wid * _BPW

        pltpu.sync_copy(h_h.at[pl.ds(base, _BPW)], hv.at[pl.ds(0, _BPW)])
        pltpu.sync_copy(t_h.at[pl.ds(base, _BPW)], tv.at[pl.ds(0, _BPW)])
        pltpu.sync_copy(r_h.at[pl.ds(base, _BPW)], rv.at[pl.ds(0, _BPW)])

        def issue(slot, hs, ts, rs):
            pltpu.async_copy(ent1_h.at[hs], ring.at[slot, 0], sem.at[slot])
            pltpu.async_copy(ent2_h.at[hs], ring.at[slot, 1], sem.at[slot])
            pltpu.async_copy(ent1_h.at[ts], ring.at[slot, 2], sem.at[slot])
            pltpu.async_copy(ent2_h.at[ts], ring.at[slot, 3], sem.at[slot])
            pltpu.async_copy(rel1_h.at[rs], ring.at[slot, 4], sem.at[slot])
            pltpu.async_copy(rel2_h.at[rs], ring.at[slot, 5], sem.at[slot])

        def drain(slot):
            pltpu.make_async_copy(
                ent1_h.at[pl.ds(0, 6)], ring.at[slot], sem.at[slot]).wait()

        def fold(slot, g):
            acc = jnp.zeros((_L,), jnp.float32)
            for k in range(_D // _L):
                ksl = pl.ds(k * _L, _L)
                x1 = ring[slot, 0, ksl]
                x2 = ring[slot, 1, ksl]
                y1 = ring[slot, 2, ksl]
                y2 = ring[slot, 3, ksl]
                w1 = ring[slot, 4, ksl]
                w2 = ring[slot, 5, ksl]
                acc = acc + y1 * (x1 * w1 - x2 * w2) + y2 * (x2 * w1 + x1 * w2)
            ps[g, pl.ds(slot * _L, _L)] = acc

        h0 = hv[pl.ds(0, _L)]
        t0 = tv[pl.ds(0, _L)]
        r0 = rv[pl.ds(0, _L)]
        for b in range(_NBUF):
            issue(b, h0[b], t0[b], r0[b])

        def group(g, carry):
            hn = hv[pl.ds((g + 1) * _NBUF, _L)]
            tn = tv[pl.ds((g + 1) * _NBUF, _L)]
            rn = rv[pl.ds((g + 1) * _NBUF, _L)]
            for b in range(_NBUF):
                drain(b)
                fold(b, g)

                @pl.when(g < _NGRP - 1)
                def _():
                    issue(b, hn[b], tn[b], rn[b])
            return carry

        lax.fori_loop(0, _NGRP, group, 0)
        pltpu.sync_copy(ps, out_h.at[pl.ds(wid * _NGRP, _NGRP)])

    return sc_kernel(ent1, ent2, rel1, rel2, h, t, r)


def _tc_loss(p2, yneg_rep):
    def tc_kernel(p_ref, y_ref, o_ref):
        t = p_ref[...] * y_ref[...]
        li = lax.broadcasted_iota(jnp.int32, (128, 128), 0)
        ci = lax.broadcasted_iota(jnp.int32, (128, 128), 1)
        fold = (li // _L == ci // _L).astype(jnp.float32)
        z = jnp.dot(t, fold, preferred_element_type=jnp.float32)
        sp = jnp.maximum(z, 0.0) + jnp.log1p(jnp.exp(-jnp.abs(z)))
        o_ref[0, 0] = jnp.sum(sp) * (1.0 / (_L * _B))

    return pl.pallas_call(
        tc_kernel,
        out_shape=jax.ShapeDtypeStruct((1, 1), jnp.float32),
        out_specs=pl.BlockSpec(memory_space=pltpu.SMEM),
    )(p2, yneg_rep)


def kernel(ent1, ent2, rel1, rel2, h, t, r, y):
    partials = _sc_partials(ent1, ent2, rel1, rel2, h, t, r)
    yneg_rep = jnp.repeat(-y, _L).reshape(_B // 8, 8 * _L)
    loss = _tc_loss(partials, yneg_rep)
    return loss[0, 0]


# speedup vs baseline: 1.5245x; 1.0008x over previous
import functools

import jax
import jax.numpy as jnp
from jax import lax
from jax.experimental import pallas as pl
from jax.experimental.pallas import tpu as pltpu
from jax.experimental.pallas import tpu_sc as plsc

_INFO = plsc.get_sparse_core_info()
_NC, _NS, _L = _INFO.num_cores, _INFO.num_subcores, _INFO.num_lanes
_NW = _NC * _NS

_B = 16384
_D = 64
_BPW = _B // _NW
_NBUF = 8
_NGRP = _BPW // _NBUF


def _sc_partials(ent1, ent2, rel1, rel2, h, t, r):
    mesh = plsc.VectorSubcoreMesh(core_axis_name="c", subcore_axis_name="s")

    @functools.partial(
        pl.kernel,
        out_type=jax.ShapeDtypeStruct((_B // _NBUF, _NBUF * _L), jnp.float32),
        mesh=mesh,
        scratch_types=[
            pltpu.VMEM((_BPW + _L,), jnp.int32),
            pltpu.VMEM((_BPW + _L,), jnp.int32),
            pltpu.VMEM((_BPW + _L,), jnp.int32),
            pltpu.VMEM((_NBUF, 6, _D), jnp.float32),
            pltpu.VMEM((_NGRP, _NBUF * _L), jnp.float32),
            pltpu.SemaphoreType.DMA((_NBUF,)),
        ],
        compiler_params=pltpu.CompilerParams(use_tc_tiling_on_sc=True),
    )
    def sc_kernel(ent1_h, ent2_h, rel1_h, rel2_h, h_h, t_h, r_h, out_h,
                  hv, tv, rv, ring, ps, sem):
        wid = lax.axis_index("s") * _NC + lax.axis_index("c")
        base = wid * _BPW

        pltpu.sync_copy(h_h.at[pl.ds(base, _BPW)], hv.at[pl.ds(0, _BPW)])
        pltpu.sync_copy(t_h.at[pl.ds(base, _BPW)], tv.at[pl.ds(0, _BPW)])
        pltpu.sync_copy(r_h.at[pl.ds(base, _BPW)], rv.at[pl.ds(0, _BPW)])

        def issue(slot, hs, ts, rs):
            pltpu.async_copy(ent1_h.at[hs], ring.at[slot, 0], sem.at[slot])
            pltpu.async_copy(ent2_h.at[hs], ring.at[slot, 1], sem.at[slot])
            pltpu.async_copy(ent1_h.at[ts], ring.at[slot, 2], sem.at[slot])
            pltpu.async_copy(ent2_h.at[ts], ring.at[slot, 3], sem.at[slot])
            pltpu.async_copy(rel1_h.at[rs], ring.at[slot, 4], sem.at[slot])
            pltpu.async_copy(rel2_h.at[rs], ring.at[slot, 5], sem.at[slot])

        def drain(slot):
            pltpu.make_async_copy(
                ent1_h.at[pl.ds(0, 6)], ring.at[slot], sem.at[slot]).wait()

        def fold(slot, g):
            acc = jnp.zeros((_L,), jnp.float32)
            for k in range(_D // _L):
                ksl = pl.ds(k * _L, _L)
                x1 = ring[slot, 0, ksl]
                x2 = ring[slot, 1, ksl]
                y1 = ring[slot, 2, ksl]
                y2 = ring[slot, 3, ksl]
                w1 = ring[slot, 4, ksl]
                w2 = ring[slot, 5, ksl]
                acc = acc + y1 * (x1 * w1 - x2 * w2) + y2 * (x2 * w1 + x1 * w2)
            ps[g, pl.ds(slot * _L, _L)] = acc

        h0 = hv[pl.ds(0, _L)]
        t0 = tv[pl.ds(0, _L)]
        r0 = rv[pl.ds(0, _L)]
        for b in range(_NBUF):
            issue(b, h0[b], t0[b], r0[b])

        def group(g, carry):
            hn = hv[pl.ds((g + 1) * _NBUF, _L)]
            tn = tv[pl.ds((g + 1) * _NBUF, _L)]
            rn = rv[pl.ds((g + 1) * _NBUF, _L)]
            for b in range(_NBUF):
                drain(b)
                fold(b, g)

                @pl.when(g < _NGRP - 1)
                def _():
                    issue(b, hn[b], tn[b], rn[b])
            return carry

        lax.fori_loop(0, _NGRP, group, 0)
        pltpu.sync_copy(ps, out_h.at[pl.ds(wid * _NGRP, _NGRP)])

    return sc_kernel(ent1, ent2, rel1, rel2, h, t, r)


def _tc_loss(p2, yneg_rep):
    def tc_kernel(p_ref, y_ref, o_ref):
        t = p_ref[...] * y_ref[...]
        li = lax.broadcasted_iota(jnp.int32, (128, 128), 0)
        ci = lax.broadcasted_iota(jnp.int32, (128, 128), 1)
        fold = (li // _L == ci // _L).astype(jnp.float32)
        z = jnp.dot(t, fold, preferred_element_type=jnp.float32)
        sp = jnp.maximum(z, 0.0) + jnp.log1p(jnp.exp(-jnp.abs(z)))
        o_ref[0, 0] = jnp.sum(sp) * (1.0 / (_L * _B))

    return pl.pallas_call(
        tc_kernel,
        out_shape=jax.ShapeDtypeStruct((1, 1), jnp.float32),
        out_specs=pl.BlockSpec(memory_space=pltpu.SMEM),
    )(p2, yneg_rep)


def kernel(ent1, ent2, rel1, rel2, h, t, r, y):
    partials = _sc_partials(ent1, ent2, rel1, rel2, h, t, r)
    yneg_rep = jnp.repeat(-y, _L).reshape(_B // 8, 8 * _L)
    loss = _tc_loss(partials, yneg_rep)
    return loss[0, 0]


# speedup vs baseline: 2.2163x; 1.4538x over previous
import functools

import jax
import jax.numpy as jnp
from jax import lax
from jax.experimental import pallas as pl
from jax.experimental.pallas import tpu as pltpu
from jax.experimental.pallas import tpu_sc as plsc

_INFO = plsc.get_sparse_core_info()
_NC, _NS, _L = _INFO.num_cores, _INFO.num_subcores, _INFO.num_lanes
_NW = _NC * _NS

_B = 16384
_D = 64
_BPW = _B // _NW
_NBUF = 8
_NGRP = _BPW // _NBUF
_ENT = 1000000
_PREP_BLK = 4096


def _tc_prep(e1t, e2t):
    def prep_kernel(a_ref, b_ref, o_ref):
        au = jax.lax.bitcast_convert_type(a_ref[...].T, jnp.uint32)
        bu = jax.lax.bitcast_convert_type(b_ref[...].T, jnp.uint32)
        half = jnp.uint32(0x8000)
        hi = jnp.uint32(0xFFFF0000)
        o_ref[...] = ((bu + half) & hi) | ((au + half) >> jnp.uint32(16))

    return pl.pallas_call(
        prep_kernel,
        grid=(pl.cdiv(_ENT, _PREP_BLK),),
        in_specs=[
            pl.BlockSpec((_D, _PREP_BLK), lambda j: (0, j)),
            pl.BlockSpec((_D, _PREP_BLK), lambda j: (0, j)),
        ],
        out_specs=pl.BlockSpec((_PREP_BLK, _D), lambda j: (j, 0)),
        out_shape=jax.ShapeDtypeStruct((_ENT, _D), jnp.uint32),
    )(e1t, e2t)


def _unpack_pair(w):
    lo = jax.lax.bitcast_convert_type(w << jnp.uint32(16), jnp.float32)
    hi = jax.lax.bitcast_convert_type(w & jnp.uint32(0xFFFF0000), jnp.float32)
    return lo, hi


def _sc_partials(ctab, rtab, h, t, r):
    mesh = plsc.VectorSubcoreMesh(core_axis_name="c", subcore_axis_name="s")

    @functools.partial(
        pl.kernel,
        out_type=jax.ShapeDtypeStruct((_B // _NBUF, _NBUF * _L), jnp.float32),
        mesh=mesh,
        scratch_types=[
            pltpu.VMEM((_BPW + _L,), jnp.int32),
            pltpu.VMEM((_BPW + _L,), jnp.int32),
            pltpu.VMEM((_BPW + _L,), jnp.int32),
            pltpu.VMEM((_NBUF, 3, _D), jnp.uint32),
            pltpu.VMEM((_NGRP, _NBUF * _L), jnp.float32),
            pltpu.SemaphoreType.DMA((_NBUF,)),
        ],
        compiler_params=pltpu.CompilerParams(use_tc_tiling_on_sc=True),
    )
    def sc_kernel(ctab_h, rtab_h, h_h, t_h, r_h, out_h,
                  hv, tv, rv, ring, ps, sem):
        wid = lax.axis_index("s") * _NC + lax.axis_index("c")
        base = wid * _BPW

        pltpu.sync_copy(h_h.at[pl.ds(base, _BPW)], hv.at[pl.ds(0, _BPW)])
        pltpu.sync_copy(t_h.at[pl.ds(base, _BPW)], tv.at[pl.ds(0, _BPW)])
        pltpu.sync_copy(r_h.at[pl.ds(base, _BPW)], rv.at[pl.ds(0, _BPW)])

        def issue(slot, hs, ts, rs):
            pltpu.async_copy(ctab_h.at[hs], ring.at[slot, 0], sem.at[slot])
            pltpu.async_copy(ctab_h.at[ts], ring.at[slot, 1], sem.at[slot])
            pltpu.async_copy(rtab_h.at[rs], ring.at[slot, 2], sem.at[slot])

        def drain(slot):
            for i in range(3):
                pltpu.make_async_copy(
                    ctab_h.at[0], ring.at[slot, i], sem.at[slot]).wait()

        def fold(slot, g):
            acc = jnp.zeros((_L,), jnp.float32)
            for k in range(_D // _L):
                ksl = pl.ds(k * _L, _L)
                x1, x2 = _unpack_pair(ring[slot, 0, ksl])
                y1, y2 = _unpack_pair(ring[slot, 1, ksl])
                w1, w2 = _unpack_pair(ring[slot, 2, ksl])
                acc = acc + y1 * (x1 * w1 - x2 * w2) + y2 * (x2 * w1 + x1 * w2)
            ps[g, pl.ds(slot * _L, _L)] = acc

        h0 = hv[pl.ds(0, _L)]
        t0 = tv[pl.ds(0, _L)]
        r0 = rv[pl.ds(0, _L)]
        for b in range(_NBUF):
            issue(b, h0[b], t0[b], r0[b])

        def group(g, carry):
            hn = hv[pl.ds((g + 1) * _NBUF, _L)]
            tn = tv[pl.ds((g + 1) * _NBUF, _L)]
            rn = rv[pl.ds((g + 1) * _NBUF, _L)]
            for b in range(_NBUF):
                drain(b)
                fold(b, g)

                @pl.when(g < _NGRP - 1)
                def _():
                    issue(b, hn[b], tn[b], rn[b])
            return carry

        lax.fori_loop(0, _NGRP, group, 0)
        pltpu.sync_copy(ps, out_h.at[pl.ds(wid * _NGRP, _NGRP)])

    return sc_kernel(ctab, rtab, h, t, r)


def _tc_loss(p2, yneg_rep):
    def tc_kernel(p_ref, y_ref, o_ref):
        t = p_ref[...] * y_ref[...]
        li = lax.broadcasted_iota(jnp.int32, (128, 128), 0)
        ci = lax.broadcasted_iota(jnp.int32, (128, 128), 1)
        fold = (li // _L == ci // _L).astype(jnp.float32)
        z = jnp.dot(t, fold, preferred_element_type=jnp.float32)
        sp = jnp.maximum(z, 0.0) + jnp.log1p(jnp.exp(-jnp.abs(z)))
        o_ref[0, 0] = jnp.sum(sp) * (1.0 / (_L * _B))

    return pl.pallas_call(
        tc_kernel,
        out_shape=jax.ShapeDtypeStruct((1, 1), jnp.float32),
        out_specs=pl.BlockSpec(memory_space=pltpu.SMEM),
    )(p2, yneg_rep)


def kernel(ent1, ent2, rel1, rel2, h, t, r, y):
    ctab = _tc_prep(ent1.T, ent2.T)
    ru1 = jax.lax.bitcast_convert_type(rel1, jnp.uint32)
    ru2 = jax.lax.bitcast_convert_type(rel2, jnp.uint32)
    half, hi = jnp.uint32(0x8000), jnp.uint32(0xFFFF0000)
    rtab = ((ru2 + half) & hi) | ((ru1 + half) >> jnp.uint32(16))
    partials = _sc_partials(ctab, rtab, h, t, r)
    yneg_rep = jnp.repeat(-y, _L).reshape(_B // 8, 8 * _L)
    loss = _tc_loss(partials, yneg_rep)
    return loss[0, 0]


# speedup vs baseline: 2.8798x; 1.2994x over previous
import functools

import jax
import jax.numpy as jnp
from jax import lax
from jax.experimental import pallas as pl
from jax.experimental.pallas import tpu as pltpu
from jax.experimental.pallas import tpu_sc as plsc

_INFO = plsc.get_sparse_core_info()
_NC, _NS, _L = _INFO.num_cores, _INFO.num_subcores, _INFO.num_lanes
_NW = _NC * _NS

_B = 16384
_D = 64
_BPW = _B // _NW
_NBUF = 8
_NGRP = _BPW // _NBUF
_ENT = 1000000
_PREP_BLK = 8192


def _tc_prep(e1t, e2t):
    def prep_kernel(a_ref, b_ref, o_ref):
        au = jax.lax.bitcast_convert_type(a_ref[...], jnp.uint32)
        bu = jax.lax.bitcast_convert_type(b_ref[...], jnp.uint32)
        half = jnp.uint32(0x8000)
        hi = jnp.uint32(0xFFFF0000)
        w = ((bu + half) & hi) | ((au + half) >> jnp.uint32(16))
        o_ref[...] = w.T

    return pl.pallas_call(
        prep_kernel,
        grid=(pl.cdiv(_ENT, _PREP_BLK),),
        in_specs=[
            pl.BlockSpec((_D, _PREP_BLK), lambda j: (0, j)),
            pl.BlockSpec((_D, _PREP_BLK), lambda j: (0, j)),
        ],
        out_specs=pl.BlockSpec((_PREP_BLK, _D), lambda j: (j, 0)),
        out_shape=jax.ShapeDtypeStruct((_ENT, _D), jnp.uint32),
    )(e1t, e2t)


def _unpack_pair(w):
    lo = jax.lax.bitcast_convert_type(w << jnp.uint32(16), jnp.float32)
    hi = jax.lax.bitcast_convert_type(w & jnp.uint32(0xFFFF0000), jnp.float32)
    return lo, hi


def _sc_partials(ctab, rtab, h, t, r):
    mesh = plsc.VectorSubcoreMesh(core_axis_name="c", subcore_axis_name="s")

    @functools.partial(
        pl.kernel,
        out_type=jax.ShapeDtypeStruct((_B // _NBUF, _NBUF * _L), jnp.float32),
        mesh=mesh,
        scratch_types=[
            pltpu.VMEM((_BPW + _L,), jnp.int32),
            pltpu.VMEM((_BPW + _L,), jnp.int32),
            pltpu.VMEM((_BPW + _L,), jnp.int32),
            pltpu.VMEM((_NBUF, 3, _D), jnp.uint32),
            pltpu.VMEM((_NGRP, _NBUF * _L), jnp.float32),
            pltpu.SemaphoreType.DMA((_NBUF,)),
        ],
        compiler_params=pltpu.CompilerParams(use_tc_tiling_on_sc=True),
    )
    def sc_kernel(ctab_h, rtab_h, h_h, t_h, r_h, out_h,
                  hv, tv, rv, ring, ps, sem):
        wid = lax.axis_index("s") * _NC + lax.axis_index("c")
        base = wid * _BPW

        pltpu.sync_copy(h_h.at[pl.ds(base, _BPW)], hv.at[pl.ds(0, _BPW)])
        pltpu.sync_copy(t_h.at[pl.ds(base, _BPW)], tv.at[pl.ds(0, _BPW)])
        pltpu.sync_copy(r_h.at[pl.ds(base, _BPW)], rv.at[pl.ds(0, _BPW)])

        def issue(slot, hs, ts, rs):
            pltpu.async_copy(ctab_h.at[hs], ring.at[slot, 0], sem.at[slot])
            pltpu.async_copy(ctab_h.at[ts], ring.at[slot, 1], sem.at[slot])
            pltpu.async_copy(rtab_h.at[rs], ring.at[slot, 2], sem.at[slot])

        def drain(slot):
            for i in range(3):
                pltpu.make_async_copy(
                    ctab_h.at[0], ring.at[slot, i], sem.at[slot]).wait()

        def fold(slot, g):
            acc = jnp.zeros((_L,), jnp.float32)
            for k in range(_D // _L):
                ksl = pl.ds(k * _L, _L)
                x1, x2 = _unpack_pair(ring[slot, 0, ksl])
                y1, y2 = _unpack_pair(ring[slot, 1, ksl])
                w1, w2 = _unpack_pair(ring[slot, 2, ksl])
                acc = acc + y1 * (x1 * w1 - x2 * w2) + y2 * (x2 * w1 + x1 * w2)
            ps[g, pl.ds(slot * _L, _L)] = acc

        h0 = hv[pl.ds(0, _L)]
        t0 = tv[pl.ds(0, _L)]
        r0 = rv[pl.ds(0, _L)]
        for b in range(_NBUF):
            issue(b, h0[b], t0[b], r0[b])

        def group(g, carry):
            hn = hv[pl.ds((g + 1) * _NBUF, _L)]
            tn = tv[pl.ds((g + 1) * _NBUF, _L)]
            rn = rv[pl.ds((g + 1) * _NBUF, _L)]
            for b in range(_NBUF):
                drain(b)
                fold(b, g)

                @pl.when(g < _NGRP - 1)
                def _():
                    issue(b, hn[b], tn[b], rn[b])
            return carry

        lax.fori_loop(0, _NGRP, group, 0)
        pltpu.sync_copy(ps, out_h.at[pl.ds(wid * _NGRP, _NGRP)])

    return sc_kernel(ctab, rtab, h, t, r)


def _tc_loss(p2, yneg_rep):
    def tc_kernel(p_ref, y_ref, o_ref):
        t = p_ref[...] * y_ref[...]
        li = lax.broadcasted_iota(jnp.int32, (128, 128), 0)
        ci = lax.broadcasted_iota(jnp.int32, (128, 128), 1)
        fold = (li // _L == ci // _L).astype(jnp.float32)
        z = jnp.dot(t, fold, preferred_element_type=jnp.float32)
        sp = jnp.maximum(z, 0.0) + jnp.log1p(jnp.exp(-jnp.abs(z)))
        o_ref[0, 0] = jnp.sum(sp) * (1.0 / (_L * _B))

    return pl.pallas_call(
        tc_kernel,
        out_shape=jax.ShapeDtypeStruct((1, 1), jnp.float32),
        out_specs=pl.BlockSpec(memory_space=pltpu.SMEM),
    )(p2, yneg_rep)


def kernel(ent1, ent2, rel1, rel2, h, t, r, y):
    ctab = _tc_prep(ent1.T, ent2.T)
    ru1 = jax.lax.bitcast_convert_type(rel1, jnp.uint32)
    ru2 = jax.lax.bitcast_convert_type(rel2, jnp.uint32)
    half, hi = jnp.uint32(0x8000), jnp.uint32(0xFFFF0000)
    rtab = ((ru2 + half) & hi) | ((ru1 + half) >> jnp.uint32(16))
    partials = _sc_partials(ctab, rtab, h, t, r)
    yneg_rep = jnp.repeat(-y, _L).reshape(_B // 8, 8 * _L)
    loss = _tc_loss(partials, yneg_rep)
    return loss[0, 0]


# speedup vs baseline: 2.9491x; 1.0241x over previous
import functools

import jax
import jax.numpy as jnp
from jax import lax
from jax.experimental import pallas as pl
from jax.experimental.pallas import tpu as pltpu
from jax.experimental.pallas import tpu_sc as plsc

_INFO = plsc.get_sparse_core_info()
_NC, _NS, _L = _INFO.num_cores, _INFO.num_subcores, _INFO.num_lanes
_NW = _NC * _NS

_B = 16384
_D = 64
_BPW = _B // _NW
_NBUF = 8
_NGRP = _BPW // _NBUF
_ENT = 1000000
_PREP_BLK = 16384


def _tc_prep(e1t, e2t):
    def prep_kernel(a_ref, b_ref, o_ref):
        au = jax.lax.bitcast_convert_type(a_ref[...], jnp.uint32)
        bu = jax.lax.bitcast_convert_type(b_ref[...], jnp.uint32)
        half = jnp.uint32(0x8000)
        hi = jnp.uint32(0xFFFF0000)
        w = ((bu + half) & hi) | ((au + half) >> jnp.uint32(16))
        o_ref[...] = w.T

    return pl.pallas_call(
        prep_kernel,
        grid=(pl.cdiv(_ENT, _PREP_BLK),),
        in_specs=[
            pl.BlockSpec((_D, _PREP_BLK), lambda j: (0, j)),
            pl.BlockSpec((_D, _PREP_BLK), lambda j: (0, j)),
        ],
        out_specs=pl.BlockSpec((_PREP_BLK, _D), lambda j: (j, 0)),
        out_shape=jax.ShapeDtypeStruct((_ENT, _D), jnp.uint32),
    )(e1t, e2t)


def _unpack_pair(w):
    lo = jax.lax.bitcast_convert_type(w << jnp.uint32(16), jnp.float32)
    hi = jax.lax.bitcast_convert_type(w & jnp.uint32(0xFFFF0000), jnp.float32)
    return lo, hi


def _sc_partials(ctab, rtab, h, t, r):
    mesh = plsc.VectorSubcoreMesh(core_axis_name="c", subcore_axis_name="s")

    @functools.partial(
        pl.kernel,
        out_type=jax.ShapeDtypeStruct((_B // _NBUF, _NBUF * _L), jnp.float32),
        mesh=mesh,
        scratch_types=[
            pltpu.VMEM((_BPW + _L,), jnp.int32),
            pltpu.VMEM((_BPW + _L,), jnp.int32),
            pltpu.VMEM((_BPW + _L,), jnp.int32),
            pltpu.VMEM((_NBUF, 3, _D), jnp.uint32),
            pltpu.VMEM((_NGRP, _NBUF * _L), jnp.float32),
            pltpu.SemaphoreType.DMA((_NBUF,)),
        ],
        compiler_params=pltpu.CompilerParams(use_tc_tiling_on_sc=True),
    )
    def sc_kernel(ctab_h, rtab_h, h_h, t_h, r_h, out_h,
                  hv, tv, rv, ring, ps, sem):
        wid = lax.axis_index("s") * _NC + lax.axis_index("c")
        base = wid * _BPW

        pltpu.sync_copy(h_h.at[pl.ds(base, _BPW)], hv.at[pl.ds(0, _BPW)])
        pltpu.sync_copy(t_h.at[pl.ds(base, _BPW)], tv.at[pl.ds(0, _BPW)])
        pltpu.sync_copy(r_h.at[pl.ds(base, _BPW)], rv.at[pl.ds(0, _BPW)])

        def issue(slot, hs, ts, rs):
            pltpu.async_copy(ctab_h.at[hs], ring.at[slot, 0], sem.at[slot])
            pltpu.async_copy(ctab_h.at[ts], ring.at[slot, 1], sem.at[slot])
            pltpu.async_copy(rtab_h.at[rs], ring.at[slot, 2], sem.at[slot])

        def drain(slot):
            for i in range(3):
                pltpu.make_async_copy(
                    ctab_h.at[0], ring.at[slot, i], sem.at[slot]).wait()

        def fold(slot, g):
            acc = jnp.zeros((_L,), jnp.float32)
            for k in range(_D // _L):
                ksl = pl.ds(k * _L, _L)
                x1, x2 = _unpack_pair(ring[slot, 0, ksl])
                y1, y2 = _unpack_pair(ring[slot, 1, ksl])
                w1, w2 = _unpack_pair(ring[slot, 2, ksl])
                acc = acc + y1 * (x1 * w1 - x2 * w2) + y2 * (x2 * w1 + x1 * w2)
            ps[g, pl.ds(slot * _L, _L)] = acc

        h0 = hv[pl.ds(0, _L)]
        t0 = tv[pl.ds(0, _L)]
        r0 = rv[pl.ds(0, _L)]
        for b in range(_NBUF):
            issue(b, h0[b], t0[b], r0[b])

        def group(g, carry):
            hn = hv[pl.ds((g + 1) * _NBUF, _L)]
            tn = tv[pl.ds((g + 1) * _NBUF, _L)]
            rn = rv[pl.ds((g + 1) * _NBUF, _L)]
            for b in range(_NBUF):
                drain(b)
                fold(b, g)

                @pl.when(g < _NGRP - 1)
                def _():
                    issue(b, hn[b], tn[b], rn[b])
            return carry

        lax.fori_loop(0, _NGRP, group, 0)
        pltpu.sync_copy(ps, out_h.at[pl.ds(wid * _NGRP, _NGRP)])

    return sc_kernel(ctab, rtab, h, t, r)


def _tc_loss(p2, yneg_rep):
    def tc_kernel(p_ref, y_ref, o_ref):
        t = p_ref[...] * y_ref[...]
        li = lax.broadcasted_iota(jnp.int32, (128, 128), 0)
        ci = lax.broadcasted_iota(jnp.int32, (128, 128), 1)
        fold = (li // _L == ci // _L).astype(jnp.float32)
        z = jnp.dot(t, fold, preferred_element_type=jnp.float32)
        sp = jnp.maximum(z, 0.0) + jnp.log1p(jnp.exp(-jnp.abs(z)))
        o_ref[0, 0] = jnp.sum(sp) * (1.0 / (_L * _B))

    return pl.pallas_call(
        tc_kernel,
        out_shape=jax.ShapeDtypeStruct((1, 1), jnp.float32),
        out_specs=pl.BlockSpec(memory_space=pltpu.SMEM),
    )(p2, yneg_rep)


def kernel(ent1, ent2, rel1, rel2, h, t, r, y):
    ctab = _tc_prep(ent1.T, ent2.T)
    ru1 = jax.lax.bitcast_convert_type(rel1, jnp.uint32)
    ru2 = jax.lax.bitcast_convert_type(rel2, jnp.uint32)
    half, hi = jnp.uint32(0x8000), jnp.uint32(0xFFFF0000)
    rtab = ((ru2 + half) & hi) | ((ru1 + half) >> jnp.uint32(16))
    partials = _sc_partials(ctab, rtab, h, t, r)
    yneg_rep = jnp.repeat(-y, _L).reshape(_B // 8, 8 * _L)
    loss = _tc_loss(partials, yneg_rep)
    return loss[0, 0]


# speedup vs baseline: 3.4562x; 1.1720x over previous
import functools

import jax
import jax.numpy as jnp
from jax import lax
from jax.experimental import pallas as pl
from jax.experimental.pallas import tpu as pltpu
from jax.experimental.pallas import tpu_sc as plsc

_INFO = plsc.get_sparse_core_info()
_NC, _NS, _L = _INFO.num_cores, _INFO.num_subcores, _INFO.num_lanes
_NW = _NC * _NS

_B = 16384
_D = 64
_BPW = _B // _NW
_NBUF = 8
_NGRP = _BPW // _NBUF
_ENT = 1000000
_PREP_BLK = 16384


def _tc_prep(e1t, e2t):
    def prep_kernel(a_ref, b_ref, o_ref):
        au = jax.lax.bitcast_convert_type(a_ref[...], jnp.uint32)
        bu = jax.lax.bitcast_convert_type(b_ref[...], jnp.uint32)
        half = jnp.uint32(0x8000)
        hi = jnp.uint32(0xFFFF0000)
        w = ((bu + half) & hi) | ((au + half) >> jnp.uint32(16))
        hb = _PREP_BLK // 2
        o_ref[...] = jnp.concatenate((w[:, :hb].T, w[:, hb:].T), axis=1)

    return pl.pallas_call(
        prep_kernel,
        grid=(pl.cdiv(_ENT, _PREP_BLK),),
        in_specs=[
            pl.BlockSpec((_D, _PREP_BLK), lambda j: (0, j)),
            pl.BlockSpec((_D, _PREP_BLK), lambda j: (0, j)),
        ],
        out_specs=pl.BlockSpec((_PREP_BLK // 2, 2 * _D), lambda j: (j, 0)),
        out_shape=jax.ShapeDtypeStruct(
            (pl.cdiv(_ENT, _PREP_BLK) * (_PREP_BLK // 2), 2 * _D), jnp.uint32),
    )(e1t, e2t)


def _unpack_pair(w):
    lo = jax.lax.bitcast_convert_type(w << jnp.uint32(16), jnp.float32)
    hi = jax.lax.bitcast_convert_type(w & jnp.uint32(0xFFFF0000), jnp.float32)
    return lo, hi


def _sc_partials(ctab, rtab, h, t, r):
    mesh = plsc.VectorSubcoreMesh(core_axis_name="c", subcore_axis_name="s")

    @functools.partial(
        pl.kernel,
        out_type=jax.ShapeDtypeStruct((_B // _NBUF, _NBUF * _L), jnp.float32),
        mesh=mesh,
        scratch_types=[
            pltpu.VMEM((_BPW + _L,), jnp.int32),
            pltpu.VMEM((_BPW + _L,), jnp.int32),
            pltpu.VMEM((_BPW + _L,), jnp.int32),
            pltpu.VMEM((_NBUF, 3, 2 * _D), jnp.uint32),
            pltpu.VMEM((_NGRP, _NBUF * _L), jnp.float32),
            pltpu.SemaphoreType.DMA((_NBUF,)),
        ],
        compiler_params=pltpu.CompilerParams(use_tc_tiling_on_sc=True),
    )
    def sc_kernel(ctab_h, rtab_h, h_h, t_h, r_h, out_h,
                  hv, tv, rv, ring, ps, sem):
        wid = lax.axis_index("s") * _NC + lax.axis_index("c")
        base = wid * _BPW

        pltpu.sync_copy(h_h.at[pl.ds(base, _BPW)], hv.at[pl.ds(0, _BPW)])
        pltpu.sync_copy(t_h.at[pl.ds(base, _BPW)], tv.at[pl.ds(0, _BPW)])
        pltpu.sync_copy(r_h.at[pl.ds(base, _BPW)], rv.at[pl.ds(0, _BPW)])

        def issue(slot, hs, ts, rs):
            hrow = ((hs >> 14) << 13) | (hs & 8191)
            trow = ((ts >> 14) << 13) | (ts & 8191)
            pltpu.async_copy(ctab_h.at[hrow], ring.at[slot, 0], sem.at[slot])
            pltpu.async_copy(ctab_h.at[trow], ring.at[slot, 1], sem.at[slot])
            pltpu.async_copy(rtab_h.at[rs >> 1], ring.at[slot, 2], sem.at[slot])

        def drain(slot):
            for i in range(3):
                pltpu.make_async_copy(
                    ctab_h.at[0], ring.at[slot, i], sem.at[slot]).wait()

        def fold(slot, g, ho, to, ro):
            acc = jnp.zeros((_L,), jnp.float32)
            for k in range(_D // _L):
                x1, x2 = _unpack_pair(ring[slot, 0, pl.ds(ho + k * _L, _L)])
                y1, y2 = _unpack_pair(ring[slot, 1, pl.ds(to + k * _L, _L)])
                w1, w2 = _unpack_pair(ring[slot, 2, pl.ds(ro + k * _L, _L)])
                acc = acc + y1 * (x1 * w1 - x2 * w2) + y2 * (x2 * w1 + x1 * w2)
            ps[g, pl.ds(slot * _L, _L)] = acc

        h0 = hv[pl.ds(0, _L)]
        t0 = tv[pl.ds(0, _L)]
        r0 = rv[pl.ds(0, _L)]
        for b in range(_NBUF):
            issue(b, h0[b], t0[b], r0[b])

        def group(g, carry):
            hn = hv[pl.ds((g + 1) * _NBUF, _L)]
            tn = tv[pl.ds((g + 1) * _NBUF, _L)]
            rn = rv[pl.ds((g + 1) * _NBUF, _L)]
            hc = hv[pl.ds(g * _NBUF, _L)]
            tc_ = tv[pl.ds(g * _NBUF, _L)]
            rc = rv[pl.ds(g * _NBUF, _L)]
            for b in range(_NBUF):
                drain(b)
                fold(b, g, ((hc[b] >> 13) & 1) * _D,
                     ((tc_[b] >> 13) & 1) * _D, (rc[b] & 1) * _D)

                @pl.when(g < _NGRP - 1)
                def _():
                    issue(b, hn[b], tn[b], rn[b])
            return carry

        lax.fori_loop(0, _NGRP, group, 0)
        pltpu.sync_copy(ps, out_h.at[pl.ds(wid * _NGRP, _NGRP)])

    return sc_kernel(ctab, rtab, h, t, r)


def _tc_loss(p2, yneg_rep):
    def tc_kernel(p_ref, y_ref, o_ref):
        t = p_ref[...] * y_ref[...]
        li = lax.broadcasted_iota(jnp.int32, (128, 128), 0)
        ci = lax.broadcasted_iota(jnp.int32, (128, 128), 1)
        fold = (li // _L == ci // _L).astype(jnp.float32)
        z = jnp.dot(t, fold, preferred_element_type=jnp.float32)
        sp = jnp.maximum(z, 0.0) + jnp.log1p(jnp.exp(-jnp.abs(z)))
        o_ref[0, 0] = jnp.sum(sp) * (1.0 / (_L * _B))

    return pl.pallas_call(
        tc_kernel,
        out_shape=jax.ShapeDtypeStruct((1, 1), jnp.float32),
        out_specs=pl.BlockSpec(memory_space=pltpu.SMEM),
    )(p2, yneg_rep)


def kernel(ent1, ent2, rel1, rel2, h, t, r, y):
    ctab = _tc_prep(ent1.T, ent2.T)
    ru1 = jax.lax.bitcast_convert_type(rel1, jnp.uint32)
    ru2 = jax.lax.bitcast_convert_type(rel2, jnp.uint32)
    half, hi = jnp.uint32(0x8000), jnp.uint32(0xFFFF0000)
    rtab = (((ru2 + half) & hi) | ((ru1 + half) >> jnp.uint32(16))).reshape(500, 128)
    partials = _sc_partials(ctab, rtab, h, t, r)
    yneg_rep = jnp.repeat(-y, _L).reshape(_B // 8, 8 * _L)
    loss = _tc_loss(partials, yneg_rep)
    return loss[0, 0]
